# baseline (speedup 1.0000x reference)
import functools

import jax
import jax.numpy as jnp
from jax.experimental import pallas as pl
from jax.experimental.pallas import tpu as pltpu

B, H, W, C = 8, 224, 224, 96
HB = 56


def _seg_block(flat_ref, x_ref, y_ref, out_ref):
    b = pl.program_id(0)
    f = flat_ref[pl.ds(b, 1), :]
    iot = jax.lax.broadcasted_iota(jnp.int32, (1, C), 1)
    m = jnp.max(f)
    cand = jnp.where(f == m, iot, jnp.int32(C))
    c = jnp.min(cand)
    oh = (iot == c).astype(jnp.float32)
    s = x_ref[0] + y_ref[0]
    out_ref[0] = jnp.sum(s * oh.reshape(1, 1, C), axis=-1)


def kernel(x, y, flat):
    grid = (B, H // HB)
    out = pl.pallas_call(
        _seg_block,
        grid=grid,
        in_specs=[
            pl.BlockSpec((B, C), lambda b, i: (0, 0)),
            pl.BlockSpec((1, HB, W, C), lambda b, i: (b, i, 0, 0)),
            pl.BlockSpec((1, HB, W, C), lambda b, i: (b, i, 0, 0)),
        ],
        out_specs=pl.BlockSpec((1, HB, W), lambda b, i: (b, i, 0)),
        out_shape=jax.ShapeDtypeStruct((B, H, W), jnp.float32),
        compiler_params=pltpu.CompilerParams(
            dimension_semantics=("parallel", "arbitrary"),
        ),
    )(flat, x, y)
    return out
